# Initial kernel scaffold; baseline (speedup 1.0000x reference)
import jax
import jax.numpy as jnp
from jax.experimental import pallas as pl


def kernel(edge_index, edge_weight, emb_weight, W1, W2):
    raise NotImplementedError("write your pallas kernel here")


# speedup vs baseline: 7.3837x; 7.3837x over previous
import functools

import jax
import jax.numpy as jnp
from jax import lax
from jax.experimental import pallas as pl
from jax.experimental.pallas import tpu as pltpu
from jax.experimental.pallas import tpu_sc as plsc

N_NODES = 10000
N_EDGES = 320000
DIM = 128

_NC = 2
_NS = 16
_NW = _NC * _NS
_EPW = N_EDGES // _NW
_C = 80
_K = _EPW // _C
_SK = 8
_KP = 128
_NSUP = _KP // _SK
_BROWS = 80
_NB = N_NODES // _BROWS


def _agg_body(row_hbm, col_hbm, w_hbm, emb_hbm, out_hbm,
              row_sb, col_sb, w_sb, gbuf0, gbuf1, sbuf, acc, sem0, sem1):
    c = lax.axis_index("c")
    s = lax.axis_index("s")
    wid = c * _NS + s

    def zfill(i, carry):
        for j in range(DIM // 16):
            sbuf[i, pl.ds(16 * j, 16)] = jnp.zeros((16,), jnp.float32)
        return carry

    lax.fori_loop(0, _BROWS, zfill, 0)

    def zblock(i, carry):
        b = s + _NS * i

        @pl.when(b < _NB)
        def _():
            off = pl.multiple_of(b * _BROWS, _BROWS)
            pltpu.sync_copy(sbuf.at[pl.ds(0, _BROWS)],
                            acc.at[pl.ds(off, _BROWS)])

        return carry

    lax.fori_loop(0, (_NB + _NS - 1) // _NS, zblock, 0)

    plsc.subcore_barrier()

    def super_body(sk, carry):
        c0 = pl.multiple_of(sk * _SK, _SK)

        pltpu.sync_copy(row_hbm.at[wid, pl.ds(c0, _SK)], row_sb)
        pltpu.sync_copy(col_hbm.at[wid, pl.ds(c0, _SK)], col_sb)
        pltpu.sync_copy(w_hbm.at[wid, pl.ds(c0, _SK)], w_sb)

        pltpu.async_copy(emb_hbm.at[col_sb.at[0]], gbuf0, sem0)
        pltpu.async_copy(emb_hbm.at[col_sb.at[1]], gbuf1, sem1)

        def half(kk, gbuf, sem):
            k = c0 + kk

            @pl.when(k < _K)
            def _():
                pltpu.make_async_copy(emb_hbm.at[col_sb.at[kk]], gbuf,
                                      sem).wait()

                def scale_group(g, carry2):
                    w16 = w_sb[kk, pl.ds(g * 16, 16)]
                    for l in range(16):
                        bw = jnp.broadcast_to(w16[l], (16,))
                        e = g * 16 + l
                        for j in range(DIM // 16):
                            sbuf[e, pl.ds(16 * j, 16)] = (
                                gbuf[e, pl.ds(16 * j, 16)] * bw)
                    return carry2

                lax.fori_loop(0, _C // 16, scale_group, 0)

                @pl.when((kk + 2 < _SK) & (k + 2 < _K))
                def _():
                    pltpu.async_copy(emb_hbm.at[col_sb.at[kk + 2]], gbuf, sem)

                pltpu.sync_copy(sbuf, acc.at[row_sb.at[kk]], add=True)

        def chunk_pair(t, carry2):
            half(2 * t, gbuf0, sem0)
            half(2 * t + 1, gbuf1, sem1)
            return carry2

        lax.fori_loop(0, _SK // 2, chunk_pair, 0)
        return carry

    lax.fori_loop(0, _NSUP, super_body, 0)

    plsc.subcore_barrier()

    def wblock(i, carry):
        b = s + _NS * i

        @pl.when(b < _NB)
        def _():
            off = pl.multiple_of(b * _BROWS, _BROWS)
            pltpu.sync_copy(acc.at[pl.ds(off, _BROWS)],
                            out_hbm.at[c, pl.ds(off, _BROWS)])

        return carry

    lax.fori_loop(0, (_NB + _NS - 1) // _NS, wblock, 0)


_aggregate = functools.partial(
    pl.kernel,
    out_type=jax.ShapeDtypeStruct((_NC, N_NODES, DIM), jnp.float32),
    mesh=plsc.VectorSubcoreMesh(core_axis_name="c", subcore_axis_name="s"),
    scratch_types=[
        pltpu.VMEM((_SK, _C), jnp.int32),
        pltpu.VMEM((_SK, _C), jnp.int32),
        pltpu.VMEM((_SK, _C), jnp.float32),
        pltpu.VMEM((_C, DIM), jnp.float32),
        pltpu.VMEM((_C, DIM), jnp.float32),
        pltpu.VMEM((_C, DIM), jnp.float32),
        pltpu.VMEM_SHARED((N_NODES, DIM), jnp.float32),
        pltpu.SemaphoreType.DMA,
        pltpu.SemaphoreType.DMA,
    ],
)(_agg_body)


_BM = 1000


def _mlp_body(p_ref, w1_ref, w2_ref, o_ref):
    x = p_ref[0] + p_ref[1]
    h = jnp.maximum(
        lax.dot_general(x, w1_ref[...], (((1,), (1,)), ((), ())),
                        preferred_element_type=jnp.float32), 0.0)
    o_ref[...] = lax.dot_general(h, w2_ref[...], (((1,), (1,)), ((), ())),
                                 preferred_element_type=jnp.float32)


def _mlp(partials, W1, W2):
    return pl.pallas_call(
        _mlp_body,
        grid=(N_NODES // _BM,),
        in_specs=[
            pl.BlockSpec((_NC, _BM, DIM), lambda i: (0, i, 0)),
            pl.BlockSpec((DIM, DIM), lambda i: (0, 0)),
            pl.BlockSpec((DIM, DIM), lambda i: (0, 0)),
        ],
        out_specs=pl.BlockSpec((_BM, DIM), lambda i: (i, 0)),
        out_shape=jax.ShapeDtypeStruct((N_NODES, DIM), jnp.float32),
    )(partials, W1, W2)


@jax.jit
def kernel(edge_index, edge_weight, emb_weight, W1, W2):
    pad = ((0, 0), (0, _KP - _K), (0, 0))
    row = jnp.pad(edge_index[0].reshape(_NW, _K, _C), pad)
    col = jnp.pad(edge_index[1].reshape(_NW, _K, _C), pad)
    w = jnp.pad(edge_weight.reshape(_NW, _K, _C), pad)
    partials = _aggregate(row, col, w, emb_weight)
    return _mlp(partials, W1, W2)
